# baseline (speedup 1.0000x reference)
import functools

import jax
import jax.numpy as jnp
from jax import lax
from jax.experimental import pallas as pl
from jax.experimental.pallas import tpu as pltpu
from jax.experimental.pallas import tpu_sc as plsc

_NUM_CLASSES = 80
_IMG_SIZE = 1024.0
_LAMBDA_COORD = 5.0
_LAMBDA_NOOBJ = 0.5

_SC_ROWS = 64
_TC_BLOCKS = 6
_SC_N = 128
_SC_CHUNKS = 4

_L1P = [
    0.01000928961813237, -0.05243753706782591, 0.1308334279841901,
    -0.22316586411920608, 0.3272257149735533, -0.4992850491225031,
    0.9999670809438583, 2.55467301950837e-07,
]
_LN2 = 0.6931471805599453


def _poly_log1p(t):
    acc = jnp.full(t.shape, _L1P[0], jnp.float32)
    for c in _L1P[1:]:
        acc = acc * t + c
    return acc


def _sc_log(x):
    bits = lax.bitcast_convert_type(x, jnp.int32)
    e = lax.shift_right_logical(bits, 23) - 127
    m = lax.bitcast_convert_type(
        (bits & 0x007FFFFF) | 0x3F800000, jnp.float32
    )
    return e.astype(jnp.float32) * _LN2 + _poly_log1p(m - 1.0)


def _softplus(z):
    return jnp.maximum(z, 0.0) + jnp.log1p(jnp.exp(-jnp.abs(z)))


def _tree_reduce(mats, op):
    while len(mats) > 1:
        nxt = [op(mats[i], mats[i + 1]) for i in range(0, len(mats) - 1, 2)]
        if len(mats) % 2:
            nxt.append(mats[-1])
        mats = nxt
    return mats[0]


def _tc_block_kernel(anc_ref, pred_ref, tgt_ref, out_ref, sel_ref, *, na, nai, nattr, r, g):
    lanes = 6 * g

    @pl.when(pl.program_id(0) == 0)
    def _build_sel():
        row = lax.broadcasted_iota(jnp.int32, (lanes, lanes), 0)
        col = lax.broadcasted_iota(jnp.int32, (lanes, lanes), 1)
        want = (row % 6) * g + row // 6
        sel_ref[...] = (want == col).astype(jnp.bfloat16)

    tmat = tgt_ref[...].reshape(na * r, lanes)
    t_hi = tmat.astype(jnp.bfloat16)
    t_lo = (tmat - t_hi.astype(jnp.float32)).astype(jnp.bfloat16)
    sel = sel_ref[...]
    dims = (((1,), (0,)), ((), ()))
    d = lax.dot_general(
        t_hi, sel, dims, preferred_element_type=jnp.float32
    ) + lax.dot_general(t_lo, sel, dims, preferred_element_type=jnp.float32)

    def attr(k):
        return d[:, k * g : (k + 1) * g].reshape(na, r, g)

    tx, ty, tw, th, t4, t5 = (attr(k) for k in range(6))
    obj = (t4 > 0.0).astype(jnp.float32)
    noobj = (t4 == 0.0).astype(jnp.float32)
    n_obj = jnp.sum(obj)
    n_noobj = jnp.sum(noobj)

    def rows(a0):
        return jnp.concatenate(
            [pred_ref[a * nattr + a0 : a * nattr + a0 + 1] for a in range(na)], axis=0
        )

    x = jax.nn.sigmoid(rows(0))
    sx = jnp.sum(obj * (x - tx) ** 2)
    y = jax.nn.sigmoid(rows(1))
    sy = jnp.sum(obj * (y - ty) ** 2)

    ew = jnp.exp(rows(2))
    eh = jnp.exp(rows(3))
    ew2 = ew * ew
    eh2 = eh * eh

    def weighted(mat, coef):
        outs = []
        for i in range(na):
            acc = coef(i, 0) * mat[0:1]
            for j in range(1, nai):
                acc = acc + coef(i, j) * mat[j : j + 1]
            outs.append(acc)
        return jnp.concatenate(outs, axis=0)

    aw = lambda i, j: anc_ref[0, i, j]
    ah = lambda i, j: anc_ref[1, i, j]
    aw2 = lambda i, j: anc_ref[0, i, j] * anc_ref[0, i, j]
    ah2 = lambda i, j: anc_ref[1, i, j] * anc_ref[1, i, j]
    fnai = float(nai)
    sw = jnp.sum(
        obj * (weighted(ew2, aw2) - 2.0 * tw * weighted(ew, aw) + fnai * tw * tw)
    )
    sh = jnp.sum(
        obj * (weighted(eh2, ah2) - 2.0 * th * weighted(eh, ah) + fnai * th * th)
    )

    zc = rows(4)
    ec = _softplus(zc) - t4 * zc
    so = jnp.sum(obj * ec)
    sn = jnp.sum(noobj * ec)

    sc = jnp.float32(0.0)
    for a in range(na):
        sum_z = None
        sum_az = None
        prod_w = None
        chunk = 8
        for c0 in range(5, nattr, chunk):
            zs = [pred_ref[a * nattr + c] for c in range(c0, min(c0 + chunk, nattr))]
            azs = [jnp.abs(z) for z in zs]
            ws = [1.0 + jnp.exp(-az) for az in azs]
            pz = _tree_reduce(zs, jnp.add)
            paz = _tree_reduce(azs, jnp.add)
            pw = _tree_reduce(ws, jnp.multiply)
            sum_z = pz if sum_z is None else sum_z + pz
            sum_az = paz if sum_az is None else sum_az + paz
            prod_w = pw if prod_w is None else prod_w * pw
        s_sum = 0.5 * (sum_z + sum_az) + jnp.log(prod_w)
        sc = sc + jnp.sum(obj[a] * (s_sum - t5[a] * sum_z))

    vals = [n_obj, n_noobj, sx, sy, sw, sh, so, sn, sc]
    for k, v in enumerate(vals):
        out_ref[0, 0, k] = v
    for k in range(len(vals), 16):
        out_ref[0, 0, k] = jnp.float32(0.0)


def _sc_body(pred_hbm, tgt_hbm, anc_hbm, out_hbm,
             pbuf, tbuf, ancb, wl0, wl1, wl2, accv, semp0, semp1, semt0, semt1):
    g = 256
    nattr = 85
    row0 = g - _SC_ROWS
    wid = lax.axis_index("s") * 2 + lax.axis_index("c")
    cpw = _SC_N * _SC_CHUNKS
    iota16 = lax.iota(jnp.int32, 16)
    zero16 = jnp.zeros((16,), jnp.int32)
    zf = jnp.zeros((16,), jnp.float32)
    semp = [semp0, semp1]
    semt = [semt0, semt1]

    def issue(k):
        l0 = wid * cpw + k * _SC_N
        row = row0 + l0 // g
        col = l0 % g
        b = k % 2
        cp = pltpu.async_copy(
            pred_hbm.at[:, pl.ds(row, 1), pl.ds(col, _SC_N)], pbuf.at[b], semp[b]
        )
        ct = pltpu.async_copy(
            tgt_hbm.at[:, pl.ds(row, 1), pl.ds(col * 6, _SC_N * 6)], tbuf.at[b],
            semt[b],
        )
        return cp, ct

    pltpu.sync_copy(anc_hbm, ancb)

    def anc_row(group, i, j):
        return ancb[pl.ds((group * 9 + i * 3 + j) * 16, 16)]

    cps = {0: issue(0)}
    accs = [zf] * 9

    for k in range(_SC_CHUNKS):
        b = k % 2
        if k + 1 < _SC_CHUNKS:
            cps[k + 1] = issue(k + 1)
        cps[k][0].wait()
        cps[k][1].wait()
        pref = pbuf.at[b]
        tref = tbuf.at[b]

        def gat(ch_vec, cell_vec, mask=None):
            return plsc.load_gather(pref, [ch_vec, zero16, cell_vec], mask=mask)

        def tgat(a_vec, lane_vec, mask=None):
            return plsc.load_gather(tref, [a_vec, zero16, lane_vec], mask=mask)

        def stage_a(gi, carry):
            (nobj, nnoobj, sx, sy, sw, sh, soc, snc, c0, c1, c2) = carry
            ds16 = pl.ds(gi * 16, 16)
            cellv = gi * 16 + iota16
            ew = [jnp.exp(pbuf[b, j * nattr + 2, 0, ds16]) for j in range(3)]
            eh = [jnp.exp(pbuf[b, j * nattr + 3, 0, ds16]) for j in range(3)]
            ew2 = [e * e for e in ew]
            eh2 = [e * e for e in eh]
            cnts = [c0, c1, c2]
            for a in range(3):
                av = jnp.full((16,), a, jnp.int32)
                lane6 = cellv * 6
                t4 = tgat(av, lane6 + 4)
                obj_m = t4 > 0.0
                objf = jnp.where(obj_m, 1.0, 0.0)
                noobjf = jnp.where(t4 == 0.0, 1.0, 0.0)
                nobj = nobj + objf
                nnoobj = nnoobj + noobjf
                z = pbuf[b, a * nattr + 4, 0, ds16]
                az = jnp.abs(z)
                v = jnp.exp(-az)
                elem = 0.5 * (z + az) + _poly_log1p(v) - t4 * z
                soc = soc + objf * elem
                snc = snc + noobjf * elem
                tx = tgat(av, lane6)
                ty = tgat(av, lane6 + 1)
                zx = pbuf[b, a * nattr + 0, 0, ds16]
                zy = pbuf[b, a * nattr + 1, 0, ds16]
                sigx = 1.0 / (1.0 + jnp.exp(-zx))
                sigy = 1.0 / (1.0 + jnp.exp(-zy))
                sx = sx + objf * (sigx - tx) * (sigx - tx)
                sy = sy + objf * (sigy - ty) * (sigy - ty)
                tw = tgat(av, lane6 + 2)
                th = tgat(av, lane6 + 3)
                s1w = anc_row(0, a, 0) * ew[0] + anc_row(0, a, 1) * ew[1] + anc_row(0, a, 2) * ew[2]
                s2w = anc_row(1, a, 0) * ew2[0] + anc_row(1, a, 1) * ew2[1] + anc_row(1, a, 2) * ew2[2]
                s1h = anc_row(2, a, 0) * eh[0] + anc_row(2, a, 1) * eh[1] + anc_row(2, a, 2) * eh[2]
                s2h = anc_row(3, a, 0) * eh2[0] + anc_row(3, a, 1) * eh2[1] + anc_row(3, a, 2) * eh2[2]
                sw = sw + objf * (s2w - 2.0 * tw * s1w + 3.0 * tw * tw)
                sh = sh + objf * (s2h - 2.0 * th * s1h + 3.0 * th * th)
                wla = (wl0, wl1, wl2)[a]
                plsc.store_compressed(
                    wla.at[pl.ds(cnts[a], 16)], cellv, mask=obj_m
                )
                cnts[a] = cnts[a] + jnp.sum(obj_m.astype(jnp.int32))
            return (nobj, nnoobj, sx, sy, sw, sh, soc, snc,
                    cnts[0], cnts[1], cnts[2])

        carry = lax.fori_loop(
            0, _SC_N // 16, stage_a,
            (accs[0], accs[1], accs[2], accs[3], accs[4], accs[5],
             accs[6], accs[7], jnp.int32(0), jnp.int32(0), jnp.int32(0)),
        )
        accs[0:8] = list(carry[0:8])
        cnts = carry[8:11]

        scls = accs[8]
        for a in range(3):
            cnt_a = cnts[a]
            av = jnp.full((16,), a, jnp.int32)

            def stage_b(t, sacc, a=a, av=av, cnt_a=cnt_a):
                lane = t * 16 + iota16
                mask = lane < cnt_a
                cellv = (wl0, wl1, wl2)[a][pl.ds(t * 16, 16)]
                t5 = tgat(av, cellv * 6 + 5, mask)

                def cls_body(c, car):
                    sz, saz, pr = car
                    chv = jnp.full((16,), a * nattr + 5, jnp.int32) + c
                    z = gat(chv, cellv, mask)
                    az = jnp.abs(z)
                    return (sz + z, saz + az, pr * (1.0 + jnp.exp(-az)))

                sz, saz, pr = lax.fori_loop(
                    0, _NUM_CLASSES, cls_body, (zf, zf, jnp.ones((16,), jnp.float32))
                )
                s_sum = 0.5 * (sz + saz) + _sc_log(pr)
                return sacc + jnp.where(mask, s_sum - t5 * sz, 0.0)

            scls = lax.fori_loop(0, (cnt_a + 15) // 16, stage_b, scls)
        accs[8] = scls

    for i in range(9):
        accv[pl.ds(i * 16, 16)] = accs[i]
    pltpu.sync_copy(accv, out_hbm.at[wid])


def _sc_partials(pred3, tgt3, ancvec):
    mesh = plsc.VectorSubcoreMesh(
        core_axis_name="c", subcore_axis_name="s", num_cores=2, num_subcores=16
    )
    fn = pl.kernel(
        _sc_body,
        out_type=jax.ShapeDtypeStruct((32, 144), jnp.float32),
        mesh=mesh,
        compiler_params=pltpu.CompilerParams(needs_layout_passes=False),
        scratch_types=[
            pltpu.VMEM((2, 255, 1, _SC_N), jnp.float32),
            pltpu.VMEM((2, 3, 1, _SC_N * 6), jnp.float32),
            pltpu.VMEM((576,), jnp.float32),
            pltpu.VMEM((_SC_N + 32,), jnp.int32),
            pltpu.VMEM((_SC_N + 32,), jnp.int32),
            pltpu.VMEM((_SC_N + 32,), jnp.int32),
            pltpu.VMEM((144,), jnp.float32),
            pltpu.SemaphoreType.DMA,
            pltpu.SemaphoreType.DMA,
            pltpu.SemaphoreType.DMA,
            pltpu.SemaphoreType.DMA,
        ],
    )
    return fn(pred3, tgt3, ancvec)


def kernel(predictions, targets, anchors):
    b, ch, g, g2 = predictions.shape
    na = targets.shape[1]
    nattr = ch // na
    nai = anchors.shape[1]
    stride = _IMG_SIZE / g
    scaled = anchors / stride
    anc = jnp.stack([scaled[:, :, 0], scaled[:, :, 1]])
    aw = scaled[:, :, 0]
    ah = scaled[:, :, 1]
    ancvec = jnp.repeat(
        jnp.concatenate([aw.reshape(-1), (aw * aw).reshape(-1),
                         ah.reshape(-1), (ah * ah).reshape(-1)]), 16
    )

    pred = predictions.reshape(ch, g, g2)
    tgt = targets.reshape(na, g, g2 * 6)

    r = (g - _SC_ROWS) // _TC_BLOCKS

    body = functools.partial(
        _tc_block_kernel, na=na, nai=nai, nattr=nattr, r=r, g=g2
    )
    sc_out = _sc_partials(pred, tgt, ancvec)

    partials = pl.pallas_call(
        body,
        grid=(_TC_BLOCKS,),
        in_specs=[
            pl.BlockSpec(memory_space=pltpu.SMEM),
            pl.BlockSpec((ch, r, g2), lambda i: (0, i, 0)),
            pl.BlockSpec((na, r, g2 * 6), lambda i: (0, i, 0)),
        ],
        out_specs=pl.BlockSpec(
            (1, 1, 16), lambda i: (i, 0, 0), memory_space=pltpu.SMEM
        ),
        out_shape=jax.ShapeDtypeStruct((_TC_BLOCKS, 1, 16), jnp.float32),
        scratch_shapes=[pltpu.VMEM((g2 * 6, g2 * 6), jnp.bfloat16)],
    )(anc, pred, tgt)

    p = jnp.sum(partials.reshape(_TC_BLOCKS, 16), axis=0)[:9] + jnp.sum(
        sc_out.reshape(32, 9, 16), axis=(0, 2)
    )
    n_obj, n_noobj = p[0], p[1]
    sx, sy, sw, sh, so, sn, sc = p[2], p[3], p[4], p[5], p[6], p[7], p[8]
    total = (
        (_LAMBDA_COORD * (sx + sy) + sw + sh + so) / n_obj
        + _LAMBDA_NOOBJ * sn / n_noobj
        + sc / (n_obj * _NUM_CLASSES)
    )
    return total


# speedup vs baseline: 1.1610x; 1.1610x over previous
import functools

import jax
import jax.numpy as jnp
from jax import lax
from jax.experimental import pallas as pl
from jax.experimental.pallas import tpu as pltpu

_NUM_CLASSES = 80
_IMG_SIZE = 1024.0
_LAMBDA_COORD = 5.0
_LAMBDA_NOOBJ = 0.5


def _softplus(z):
    return jnp.maximum(z, 0.0) + jnp.log1p(jnp.exp(-jnp.abs(z)))


def _tree_reduce(mats, op):
    while len(mats) > 1:
        nxt = [op(mats[i], mats[i + 1]) for i in range(0, len(mats) - 1, 2)]
        if len(mats) % 2:
            nxt.append(mats[-1])
        mats = nxt
    return mats[0]


def _block_kernel(anc_ref, pred_ref, tgt_ref, out_ref, sel_ref, *, na, nai, nattr, r, g):
    lanes = 6 * g

    @pl.when(pl.program_id(0) == 0)
    def _build_sel():
        row = lax.broadcasted_iota(jnp.int32, (lanes, lanes), 0)
        col = lax.broadcasted_iota(jnp.int32, (lanes, lanes), 1)
        want = (row % 6) * g + row // 6
        sel_ref[...] = (want == col).astype(jnp.bfloat16)

    tmat = tgt_ref[...].reshape(na * r, lanes)
    t_hi = tmat.astype(jnp.bfloat16)
    t_lo = (tmat - t_hi.astype(jnp.float32)).astype(jnp.bfloat16)
    sel = sel_ref[...]
    dims = (((1,), (0,)), ((), ()))
    d = lax.dot_general(
        t_hi, sel, dims, preferred_element_type=jnp.float32
    ) + lax.dot_general(t_lo, sel, dims, preferred_element_type=jnp.float32)

    def attr(k):
        return d[:, k * g : (k + 1) * g].reshape(na, r, g)

    tx, ty, tw, th, t4, t5 = (attr(k) for k in range(6))
    obj = (t4 > 0.0).astype(jnp.float32)
    noobj = (t4 == 0.0).astype(jnp.float32)
    n_obj = jnp.sum(obj)
    n_noobj = jnp.sum(noobj)

    def rows(a0):
        return jnp.concatenate(
            [pred_ref[0, a * nattr + a0 : a * nattr + a0 + 1] for a in range(na)],
            axis=0,
        )

    x = jax.nn.sigmoid(rows(0))
    sx = jnp.sum(obj * (x - tx) ** 2)
    y = jax.nn.sigmoid(rows(1))
    sy = jnp.sum(obj * (y - ty) ** 2)

    ew = jnp.exp(rows(2))
    eh = jnp.exp(rows(3))
    ew2 = ew * ew
    eh2 = eh * eh

    def weighted(mat, coef):
        outs = []
        for i in range(na):
            acc = coef(i, 0) * mat[0:1]
            for j in range(1, nai):
                acc = acc + coef(i, j) * mat[j : j + 1]
            outs.append(acc)
        return jnp.concatenate(outs, axis=0)

    aw = lambda i, j: anc_ref[0, i, j]
    ah = lambda i, j: anc_ref[1, i, j]
    aw2 = lambda i, j: anc_ref[0, i, j] * anc_ref[0, i, j]
    ah2 = lambda i, j: anc_ref[1, i, j] * anc_ref[1, i, j]
    fnai = float(nai)
    sw = jnp.sum(
        obj * (weighted(ew2, aw2) - 2.0 * tw * weighted(ew, aw) + fnai * tw * tw)
    )
    sh = jnp.sum(
        obj * (weighted(eh2, ah2) - 2.0 * th * weighted(eh, ah) + fnai * th * th)
    )

    zc = rows(4)
    ec = _softplus(zc) - t4 * zc
    so = jnp.sum(obj * ec)
    sn = jnp.sum(noobj * ec)

    sc = jnp.float32(0.0)
    for a in range(na):
        sum_z = None
        sum_az = None
        prod_w = None
        chunk = 8
        for c0 in range(5, nattr, chunk):
            zs = [pred_ref[0, a * nattr + c] for c in range(c0, min(c0 + chunk, nattr))]
            azs = [jnp.abs(z) for z in zs]
            ws = [1.0 + jnp.exp(-az) for az in azs]
            pz = _tree_reduce(zs, jnp.add)
            paz = _tree_reduce(azs, jnp.add)
            pw = _tree_reduce(ws, jnp.multiply)
            sum_z = pz if sum_z is None else sum_z + pz
            sum_az = paz if sum_az is None else sum_az + paz
            prod_w = pw if prod_w is None else prod_w * pw
        s_sum = 0.5 * (sum_z + sum_az) + jnp.log(prod_w)
        sc = sc + jnp.sum(obj[a] * (s_sum - t5[a] * sum_z))

    vals = [n_obj, n_noobj, sx, sy, sw, sh, so, sn, sc]
    for k, v in enumerate(vals):
        out_ref[0, 0, k] = v
    for k in range(len(vals), 16):
        out_ref[0, 0, k] = jnp.float32(0.0)


def kernel(predictions, targets, anchors):
    b, ch, g, g2 = predictions.shape
    na = targets.shape[1]
    nattr = ch // na
    nai = anchors.shape[1]
    stride = _IMG_SIZE / g
    scaled = anchors / stride
    anc = jnp.stack([scaled[:, :, 0], scaled[:, :, 1]])

    tgt = targets.reshape(na, g, g2 * 6)

    nblocks = 8
    r = g // nblocks

    body = functools.partial(
        _block_kernel, na=na, nai=nai, nattr=nattr, r=r, g=g2
    )
    partials = pl.pallas_call(
        body,
        grid=(nblocks,),
        in_specs=[
            pl.BlockSpec(memory_space=pltpu.SMEM),
            pl.BlockSpec((1, ch, r, g2), lambda i: (0, 0, i, 0)),
            pl.BlockSpec((na, r, g2 * 6), lambda i: (0, i, 0)),
        ],
        out_specs=pl.BlockSpec(
            (1, 1, 16), lambda i: (i, 0, 0), memory_space=pltpu.SMEM
        ),
        out_shape=jax.ShapeDtypeStruct((nblocks, 1, 16), jnp.float32),
        scratch_shapes=[pltpu.VMEM((g2 * 6, g2 * 6), jnp.bfloat16)],
    )(anc, predictions, tgt)

    p = jnp.sum(partials.reshape(nblocks, 16), axis=0)
    n_obj, n_noobj = p[0], p[1]
    sx, sy, sw, sh, so, sn, sc = p[2], p[3], p[4], p[5], p[6], p[7], p[8]
    total = (
        (_LAMBDA_COORD * (sx + sy) + sw + sh + so) / n_obj
        + _LAMBDA_NOOBJ * sn / n_noobj
        + sc / (n_obj * _NUM_CLASSES)
    )
    return total
